# initial kernel scaffold (unmeasured)
import jax
import jax.numpy as jnp
from jax import lax
from jax.experimental import pallas as pl
from jax.experimental.pallas import tpu as pltpu

N_DEV = 4
N_TOK = 2048
D_IN = 512
D_OUT = 1024
N_EXP = 32
EXP_PER_DEV = N_EXP // N_DEV
CAP = 51


def kernel(x, router_W, route_idx, expert_W):
    e = route_idx[:, 0].astype(jnp.int32)
    onehot = e[:, None] == jnp.arange(N_EXP, dtype=jnp.int32)
    pos = jnp.cumsum(onehot.astype(jnp.int32), axis=0)
    my_pos = jnp.sum(pos * onehot, axis=1)
    assign = jnp.where(my_pos <= CAP, e, -1)[:, None]

    x_bf = x.astype(jnp.bfloat16)
    w_bf = expert_W.astype(jnp.bfloat16)

    def body(x_ref, assign_ref, w_ref, out_ref, comm_ref, send_sems, recv_sems):
        my = lax.axis_index("i")
        left = (my - 1) % N_DEV
        right = (my + 1) % N_DEV

        barrier = pltpu.get_barrier_semaphore()
        for nbr in (left, right):
            pl.semaphore_signal(
                barrier, inc=1, device_id=(nbr,),
                device_id_type=pl.DeviceIdType.MESH,
            )
        pl.semaphore_wait(barrier, 2)

        xv = x_ref[:, :]
        av = assign_ref[:, :]
        acc = jnp.zeros((N_TOK, D_OUT), jnp.float32)
        for le in range(EXP_PER_DEV):
            ge = my * EXP_PER_DEV + le
            xm = jnp.where(av == ge, xv, jnp.zeros_like(xv))
            acc = acc + jnp.dot(
                xm, w_ref[le], preferred_element_type=jnp.float32
            )
        out_ref[:, :] = acc
        comm_ref[0, :, :] = acc.astype(jnp.bfloat16)

        for h in range(N_DEV - 1):
            rdma = pltpu.make_async_remote_copy(
                src_ref=comm_ref.at[h],
                dst_ref=comm_ref.at[h + 1],
                send_sem=send_sems.at[h],
                recv_sem=recv_sems.at[h],
                device_id=(right,),
                device_id_type=pl.DeviceIdType.MESH,
            )
            rdma.start()
            rdma.wait()
            out_ref[:, :] = out_ref[:, :] + comm_ref[h + 1, :, :].astype(
                jnp.float32
            )

    return pl.pallas_call(
        body,
        out_shape=jax.ShapeDtypeStruct((N_TOK, D_OUT), jnp.float32),
        in_specs=[
            pl.BlockSpec(memory_space=pltpu.VMEM),
            pl.BlockSpec(memory_space=pltpu.VMEM),
            pl.BlockSpec(memory_space=pltpu.VMEM),
        ],
        out_specs=pl.BlockSpec(memory_space=pltpu.VMEM),
        scratch_shapes=[
            pltpu.VMEM((N_DEV, N_TOK, D_OUT), jnp.bfloat16),
            pltpu.SemaphoreType.DMA((N_DEV - 1,)),
            pltpu.SemaphoreType.DMA((N_DEV - 1,)),
        ],
        compiler_params=pltpu.CompilerParams(collective_id=0),
    )(x_bf, assign, w_bf)


# baseline (device time: 203169 ns/iter reference)
import jax
import jax.numpy as jnp
from jax import lax
from jax.experimental import pallas as pl
from jax.experimental.pallas import tpu as pltpu

N_DEV = 4
N_TOK = 2048
D_IN = 512
D_OUT = 1024
N_EXP = 32
EXP_PER_DEV = N_EXP // N_DEV
CAP = 51


def kernel(x, router_W, route_idx, expert_W):
    e = route_idx[:, 0].astype(jnp.int32)
    onehot = e[:, None] == jnp.arange(N_EXP, dtype=jnp.int32)
    pos = jnp.cumsum(onehot.astype(jnp.int32), axis=0)
    my_pos = jnp.sum(pos * onehot, axis=1)
    assign = jnp.where(my_pos <= CAP, e, -1)[:, None]

    x_bf = x.astype(jnp.bfloat16)
    w_bf = expert_W.astype(jnp.bfloat16)

    def body(x_ref, assign_ref, w_ref, out_ref, comm_ref, send_sems, recv_sems):
        my = lax.axis_index("i")
        left = (my - 1) % N_DEV
        right = (my + 1) % N_DEV

        barrier = pltpu.get_barrier_semaphore()
        for nbr in (left, right):
            pl.semaphore_signal(
                barrier, inc=1, device_id=(nbr,),
                device_id_type=pl.DeviceIdType.MESH,
            )
        pl.semaphore_wait(barrier, 2)

        xv = x_ref[:, :]
        av = assign_ref[:, :]
        acc = jnp.zeros((N_TOK, D_OUT), jnp.float32)
        for le in range(EXP_PER_DEV):
            ge = my * EXP_PER_DEV + le
            xm = jnp.where(av == ge, xv, jnp.zeros_like(xv))
            acc = acc + jnp.dot(
                xm, w_ref[le], preferred_element_type=jnp.float32
            )
        out_ref[:, :] = acc
        comm_ref[0, :, :] = acc.astype(jnp.bfloat16)

        for h in range(N_DEV - 1):
            rdma = pltpu.make_async_remote_copy(
                src_ref=comm_ref.at[h],
                dst_ref=comm_ref.at[h + 1],
                send_sem=send_sems.at[h],
                recv_sem=recv_sems.at[h],
                device_id=(right,),
                device_id_type=pl.DeviceIdType.MESH,
            )
            rdma.start()
            rdma.wait()
            out_ref[:, :] = out_ref[:, :] + comm_ref[h + 1, :, :].astype(
                jnp.float32
            )

    return pl.pallas_call(
        body,
        out_shape=jax.ShapeDtypeStruct((N_TOK, D_OUT), jnp.float32),
        in_specs=[
            pl.BlockSpec(memory_space=pltpu.VMEM),
            pl.BlockSpec(memory_space=pltpu.VMEM),
            pl.BlockSpec(memory_space=pltpu.VMEM),
        ],
        out_specs=pl.BlockSpec(memory_space=pltpu.VMEM),
        scratch_shapes=[
            pltpu.VMEM((N_DEV, N_TOK, D_OUT), jnp.bfloat16),
            pltpu.SemaphoreType.DMA((N_DEV - 1,)),
            pltpu.SemaphoreType.DMA((N_DEV - 1,)),
        ],
        compiler_params=pltpu.CompilerParams(
            collective_id=0, vmem_limit_bytes=100 * 1024 * 1024
        ),
    )(x_bf, assign, w_bf)


# device time: 87258 ns/iter; 2.3284x vs baseline; 2.3284x over previous
import jax
import jax.numpy as jnp
from jax import lax
from jax.experimental import pallas as pl
from jax.experimental.pallas import tpu as pltpu

N_DEV = 4
N_TOK = 2048
D_IN = 512
D_OUT = 1024
N_EXP = 32
EXP_PER_DEV = N_EXP // N_DEV
CAP = 51
SLOTS = 64
ROWS_PER_DEV = EXP_PER_DEV * SLOTS


def kernel(x, router_W, route_idx, expert_W):
    e = route_idx[:, 0].astype(jnp.int32)
    onehot = e[:, None] == jnp.arange(N_EXP, dtype=jnp.int32)
    pos = jnp.cumsum(onehot.astype(jnp.int32), axis=0)
    my_pos = jnp.sum(pos * onehot, axis=1)
    kept = my_pos <= CAP
    slot = jnp.where(kept, e * SLOTS + (my_pos - 1), 0)

    my = lax.axis_index("i")
    x_bf = x.astype(jnp.bfloat16)
    w_bf = expert_W.astype(jnp.bfloat16)

    local = kept & (e >= my * EXP_PER_DEV) & (e < (my + 1) * EXP_PER_DEV)
    lslot = jnp.where(local, slot - my * ROWS_PER_DEV, jnp.int32(1 << 30))
    tok4slot = (
        jnp.zeros((ROWS_PER_DEV,), jnp.int32)
        .at[lslot]
        .set(jnp.arange(N_TOK, dtype=jnp.int32), mode="drop")
    )
    cx = x_bf[tok4slot]

    def body(cx_ref, w_ref, out_ref, send_sems, recv_sems):
        me = lax.axis_index("i")

        barrier = pltpu.get_barrier_semaphore()
        for k in range(1, N_DEV):
            pl.semaphore_signal(
                barrier, inc=1, device_id=(lax.rem(me + k, N_DEV),),
                device_id_type=pl.DeviceIdType.MESH,
            )
        pl.semaphore_wait(barrier, N_DEV - 1)

        parts = [
            jnp.dot(
                cx_ref[le * SLOTS:(le + 1) * SLOTS, :],
                w_ref[le],
                preferred_element_type=jnp.float32,
            ).astype(jnp.bfloat16)
            for le in range(EXP_PER_DEV)
        ]
        mine = jnp.concatenate(parts, axis=0)
        row0 = me * ROWS_PER_DEV
        out_ref[pl.ds(row0, ROWS_PER_DEV), :] = mine

        rdmas = []
        for k in range(1, N_DEV):
            rdma = pltpu.make_async_remote_copy(
                src_ref=out_ref.at[pl.ds(row0, ROWS_PER_DEV), :],
                dst_ref=out_ref.at[pl.ds(row0, ROWS_PER_DEV), :],
                send_sem=send_sems.at[k - 1],
                recv_sem=recv_sems.at[k - 1],
                device_id=(lax.rem(me + k, N_DEV),),
                device_id_type=pl.DeviceIdType.MESH,
            )
            rdma.start()
            rdmas.append(rdma)
        for rdma in rdmas:
            rdma.wait()

    compact = pl.pallas_call(
        body,
        out_shape=jax.ShapeDtypeStruct((N_DEV * ROWS_PER_DEV, D_OUT), jnp.bfloat16),
        in_specs=[
            pl.BlockSpec(memory_space=pltpu.VMEM),
            pl.BlockSpec(memory_space=pltpu.VMEM),
        ],
        out_specs=pl.BlockSpec(memory_space=pltpu.VMEM),
        scratch_shapes=[
            pltpu.SemaphoreType.DMA((N_DEV - 1,)),
            pltpu.SemaphoreType.DMA((N_DEV - 1,)),
        ],
        compiler_params=pltpu.CompilerParams(
            collective_id=0, vmem_limit_bytes=100 * 1024 * 1024
        ),
    )(cx, w_bf)

    gathered = compact[slot]
    return jnp.where(kept[:, None], gathered.astype(jnp.float32), 0.0)


# device time: 76913 ns/iter; 2.6415x vs baseline; 1.1345x over previous
import jax
import jax.numpy as jnp
from jax import lax
from jax.experimental import pallas as pl
from jax.experimental.pallas import tpu as pltpu

N_DEV = 4
N_TOK = 2048
D_IN = 512
D_OUT = 1024
N_EXP = 32
EXP_PER_DEV = N_EXP // N_DEV
CAP = 51
SLOTS = 64
ROWS_PER_DEV = EXP_PER_DEV * SLOTS


def kernel(x, router_W, route_idx, expert_W):
    e = route_idx[:, 0].astype(jnp.int32)
    onehot = e[:, None] == jnp.arange(N_EXP, dtype=jnp.int32)
    pos = jnp.cumsum(onehot.astype(jnp.int32), axis=0)
    my_pos = jnp.sum(pos * onehot, axis=1)
    kept = my_pos <= CAP
    slot = jnp.where(kept, e * SLOTS + (my_pos - 1), -1)

    my = lax.axis_index("i")
    local = kept & (e >= my * EXP_PER_DEV) & (e < (my + 1) * EXP_PER_DEV)
    lslot = jnp.where(local, slot - my * ROWS_PER_DEV, jnp.int32(1 << 30))
    tok4slot = (
        jnp.zeros((ROWS_PER_DEV,), jnp.int32)
        .at[lslot]
        .set(jnp.arange(N_TOK, dtype=jnp.int32), mode="drop")
    )
    cx = x[tok4slot]

    def body(cx_ref, w_ref, slot_ref, out_ref, comm_ref, send_sems, recv_sems):
        me = lax.axis_index("i")

        barrier = pltpu.get_barrier_semaphore()
        for k in range(1, N_DEV):
            pl.semaphore_signal(
                barrier, inc=1, device_id=(lax.rem(me + k, N_DEV),),
                device_id_type=pl.DeviceIdType.MESH,
            )
        pl.semaphore_wait(barrier, N_DEV - 1)

        cxv = cx_ref[:, :].astype(jnp.bfloat16)
        parts = [
            jnp.dot(
                cxv[le * SLOTS:(le + 1) * SLOTS, :],
                w_ref[le].astype(jnp.bfloat16),
                preferred_element_type=jnp.float32,
            ).astype(jnp.bfloat16)
            for le in range(EXP_PER_DEV)
        ]
        mine = jnp.concatenate(parts, axis=0)
        row0 = me * ROWS_PER_DEV
        comm_ref[pl.ds(row0, ROWS_PER_DEV), :] = mine

        rdmas = []
        for k in range(1, N_DEV):
            rdma = pltpu.make_async_remote_copy(
                src_ref=comm_ref.at[pl.ds(row0, ROWS_PER_DEV), :],
                dst_ref=comm_ref.at[pl.ds(row0, ROWS_PER_DEV), :],
                send_sem=send_sems.at[k - 1],
                recv_sem=recv_sems.at[k - 1],
                device_id=(lax.rem(me + k, N_DEV),),
                device_id_type=pl.DeviceIdType.MESH,
            )
            rdma.start()
            rdmas.append(rdma)

        col = lax.broadcasted_iota(jnp.int32, (N_TOK, ROWS_PER_DEV), 1)

        def pblk(m):
            return (slot_ref[:, :] == col + m * ROWS_PER_DEV).astype(
                jnp.bfloat16
            )

        acc = jnp.dot(pblk(me), mine, preferred_element_type=jnp.float32)
        for k in range(1, N_DEV):
            rdmas[k - 1].wait()
            m = lax.rem(me - k + N_DEV, N_DEV)
            cblk = comm_ref[pl.ds(m * ROWS_PER_DEV, ROWS_PER_DEV), :]
            acc = acc + jnp.dot(
                pblk(m), cblk, preferred_element_type=jnp.float32
            )
        out_ref[:, :] = acc

    return pl.pallas_call(
        body,
        out_shape=jax.ShapeDtypeStruct((N_TOK, D_OUT), jnp.float32),
        in_specs=[
            pl.BlockSpec(memory_space=pltpu.VMEM),
            pl.BlockSpec(memory_space=pltpu.VMEM),
            pl.BlockSpec(memory_space=pltpu.VMEM),
        ],
        out_specs=pl.BlockSpec(memory_space=pltpu.VMEM),
        scratch_shapes=[
            pltpu.VMEM((N_DEV * ROWS_PER_DEV, D_OUT), jnp.bfloat16),
            pltpu.SemaphoreType.DMA((N_DEV - 1,)),
            pltpu.SemaphoreType.DMA((N_DEV - 1,)),
        ],
        compiler_params=pltpu.CompilerParams(
            collective_id=0, vmem_limit_bytes=100 * 1024 * 1024
        ),
    )(cx, expert_W, slot[:, None])


# device time: 59715 ns/iter; 3.4023x vs baseline; 1.2880x over previous
import jax
import jax.numpy as jnp
from jax import lax
from jax.experimental import pallas as pl
from jax.experimental.pallas import tpu as pltpu

N_DEV = 4
N_TOK = 2048
D_IN = 512
D_OUT = 1024
N_EXP = 32
EXP_PER_DEV = N_EXP // N_DEV
CAP = 51
SLOTS = 64
ROWS_PER_DEV = EXP_PER_DEV * SLOTS


def kernel(x, router_W, route_idx, expert_W):
    e = route_idx[:, 0].astype(jnp.int32)
    onehot = (e[:, None] == jnp.arange(N_EXP, dtype=jnp.int32)).astype(
        jnp.bfloat16
    )
    tri = jnp.tri(N_TOK, dtype=jnp.bfloat16)
    pos = jnp.dot(tri, onehot, preferred_element_type=jnp.float32)
    my_pos = jnp.sum(pos * onehot.astype(jnp.float32), axis=1).astype(
        jnp.int32
    )
    kept = my_pos <= CAP
    slot = jnp.where(kept, e * SLOTS + (my_pos - 1), -1)

    my = lax.axis_index("i")
    local = kept & (e >= my * EXP_PER_DEV) & (e < (my + 1) * EXP_PER_DEV)
    lslot = jnp.where(local, slot - my * ROWS_PER_DEV, -1)
    dt = (
        lslot[:, None] == jnp.arange(ROWS_PER_DEV, dtype=jnp.int32)[None, :]
    ).astype(jnp.bfloat16)

    def body(x_ref, dt_ref, slot_ref, w_ref, out_ref, comm_ref,
             send_sems, recv_sems):
        me = lax.axis_index("i")

        barrier = pltpu.get_barrier_semaphore()
        for k in range(1, N_DEV):
            pl.semaphore_signal(
                barrier, inc=1, device_id=(lax.rem(me + k, N_DEV),),
                device_id_type=pl.DeviceIdType.MESH,
            )
        pl.semaphore_wait(barrier, N_DEV - 1)

        xbf = x_ref[:, :].astype(jnp.bfloat16)
        cx = lax.dot_general(
            dt_ref[:, :], xbf, (((0,), (0,)), ((), ())),
            preferred_element_type=jnp.float32,
        ).astype(jnp.bfloat16)

        parts = [
            jnp.dot(
                cx[le * SLOTS:(le + 1) * SLOTS, :],
                w_ref[le].astype(jnp.bfloat16),
                preferred_element_type=jnp.float32,
            ).astype(jnp.bfloat16)
            for le in range(EXP_PER_DEV)
        ]
        mine = jnp.concatenate(parts, axis=0)
        row0 = me * ROWS_PER_DEV
        comm_ref[pl.ds(row0, ROWS_PER_DEV), :] = mine

        rdmas = []
        for k in range(1, N_DEV):
            rdma = pltpu.make_async_remote_copy(
                src_ref=comm_ref.at[pl.ds(row0, ROWS_PER_DEV), :],
                dst_ref=comm_ref.at[pl.ds(row0, ROWS_PER_DEV), :],
                send_sem=send_sems.at[k - 1],
                recv_sem=recv_sems.at[k - 1],
                device_id=(lax.rem(me + k, N_DEV),),
                device_id_type=pl.DeviceIdType.MESH,
            )
            rdma.start()
            rdmas.append(rdma)

        col = lax.broadcasted_iota(jnp.int32, (N_TOK, ROWS_PER_DEV), 1)

        def pblk(m):
            return (slot_ref[:, :] == col + m * ROWS_PER_DEV).astype(
                jnp.bfloat16
            )

        acc = jnp.dot(pblk(me), mine, preferred_element_type=jnp.float32)
        for k in range(1, N_DEV):
            rdmas[k - 1].wait()
            m = lax.rem(me - k + N_DEV, N_DEV)
            cblk = comm_ref[pl.ds(m * ROWS_PER_DEV, ROWS_PER_DEV), :]
            acc = acc + jnp.dot(
                pblk(m), cblk, preferred_element_type=jnp.float32
            )
        out_ref[:, :] = acc

    return pl.pallas_call(
        body,
        out_shape=jax.ShapeDtypeStruct((N_TOK, D_OUT), jnp.float32),
        in_specs=[
            pl.BlockSpec(memory_space=pltpu.VMEM),
            pl.BlockSpec(memory_space=pltpu.VMEM),
            pl.BlockSpec(memory_space=pltpu.VMEM),
            pl.BlockSpec(memory_space=pltpu.VMEM),
        ],
        out_specs=pl.BlockSpec(memory_space=pltpu.VMEM),
        scratch_shapes=[
            pltpu.VMEM((N_DEV * ROWS_PER_DEV, D_OUT), jnp.bfloat16),
            pltpu.SemaphoreType.DMA((N_DEV - 1,)),
            pltpu.SemaphoreType.DMA((N_DEV - 1,)),
        ],
        compiler_params=pltpu.CompilerParams(
            collective_id=0, vmem_limit_bytes=100 * 1024 * 1024
        ),
    )(x, dt, slot[:, None], expert_W)


# device time: 55603 ns/iter; 3.6539x vs baseline; 1.0740x over previous
import jax
import jax.numpy as jnp
from jax import lax
from jax.experimental import pallas as pl
from jax.experimental.pallas import tpu as pltpu

N_DEV = 4
N_TOK = 2048
D_IN = 512
D_OUT = 1024
N_EXP = 32
EXP_PER_DEV = N_EXP // N_DEV
CAP = 51
SLOTS = 64
ROWS_PER_DEV = EXP_PER_DEV * SLOTS
CHUNKS = 2
CHUNK_ROWS = ROWS_PER_DEV // CHUNKS


def kernel(x, router_W, route_idx, expert_W):
    e = route_idx[:, 0].astype(jnp.int32)
    onehot = (e[:, None] == jnp.arange(N_EXP, dtype=jnp.int32)).astype(
        jnp.bfloat16
    )
    tri = jnp.tri(N_TOK, dtype=jnp.bfloat16)
    pos = jnp.dot(tri, onehot, preferred_element_type=jnp.float32)
    my_pos = jnp.sum(pos * onehot.astype(jnp.float32), axis=1).astype(
        jnp.int32
    )
    kept = my_pos <= CAP
    slot = jnp.where(kept, e * SLOTS + (my_pos - 1), -1)

    my = lax.axis_index("i")
    local = kept & (e >= my * EXP_PER_DEV) & (e < (my + 1) * EXP_PER_DEV)
    lslot = jnp.where(local, slot - my * ROWS_PER_DEV, -1)
    dt = (
        lslot[:, None] == jnp.arange(ROWS_PER_DEV, dtype=jnp.int32)[None, :]
    ).astype(jnp.bfloat16)

    def body(x_ref, dt_ref, slot_ref, w_ref, out_ref, comm_ref,
             send_sems, recv_sems):
        me = lax.axis_index("i")

        barrier = pltpu.get_barrier_semaphore()
        for k in range(1, N_DEV):
            pl.semaphore_signal(
                barrier, inc=1, device_id=(lax.rem(me + k, N_DEV),),
                device_id_type=pl.DeviceIdType.MESH,
            )
        pl.semaphore_wait(barrier, N_DEV - 1)

        xbf = x_ref[:, :].astype(jnp.bfloat16)
        cx = lax.dot_general(
            dt_ref[:, :], xbf, (((0,), (0,)), ((), ())),
            preferred_element_type=jnp.float32,
        ).astype(jnp.bfloat16)

        parts = [
            jnp.dot(
                cx[le * SLOTS:(le + 1) * SLOTS, :],
                w_ref[le].astype(jnp.bfloat16),
                preferred_element_type=jnp.float32,
            ).astype(jnp.bfloat16)
            for le in range(EXP_PER_DEV)
        ]
        mine = jnp.concatenate(parts, axis=0)
        row0 = me * ROWS_PER_DEV
        comm_ref[pl.ds(row0, ROWS_PER_DEV), :] = mine

        rdmas = []
        for k in range(1, N_DEV):
            for c in range(CHUNKS):
                src = pl.ds(row0 + c * CHUNK_ROWS, CHUNK_ROWS)
                idx = (k - 1) * CHUNKS + c
                rdma = pltpu.make_async_remote_copy(
                    src_ref=comm_ref.at[src, :],
                    dst_ref=comm_ref.at[src, :],
                    send_sem=send_sems.at[idx],
                    recv_sem=recv_sems.at[idx],
                    device_id=(lax.rem(me + k, N_DEV),),
                    device_id_type=pl.DeviceIdType.MESH,
                )
                rdma.start()
                rdmas.append(rdma)

        colw = lax.broadcasted_iota(jnp.int32, (N_TOK, ROWS_PER_DEV), 1)
        colc = lax.broadcasted_iota(jnp.int32, (N_TOK, CHUNK_ROWS), 1)

        pmine = (slot_ref[:, :] == colw + row0).astype(jnp.bfloat16)
        acc = jnp.dot(pmine, mine, preferred_element_type=jnp.float32)
        for k in range(1, N_DEV):
            m = lax.rem(me - k + N_DEV, N_DEV)
            for c in range(CHUNKS):
                rdmas[(k - 1) * CHUNKS + c].wait()
                r0 = m * ROWS_PER_DEV + c * CHUNK_ROWS
                cblk = comm_ref[pl.ds(r0, CHUNK_ROWS), :]
                pblk = (slot_ref[:, :] == colc + r0).astype(jnp.bfloat16)
                acc = acc + jnp.dot(
                    pblk, cblk, preferred_element_type=jnp.float32
                )
        out_ref[:, :] = acc.astype(jnp.bfloat16)

    return pl.pallas_call(
        body,
        out_shape=jax.ShapeDtypeStruct((N_TOK, D_OUT), jnp.bfloat16),
        in_specs=[
            pl.BlockSpec(memory_space=pltpu.VMEM),
            pl.BlockSpec(memory_space=pltpu.VMEM),
            pl.BlockSpec(memory_space=pltpu.VMEM),
            pl.BlockSpec(memory_space=pltpu.VMEM),
        ],
        out_specs=pl.BlockSpec(memory_space=pltpu.VMEM),
        scratch_shapes=[
            pltpu.VMEM((N_DEV * ROWS_PER_DEV, D_OUT), jnp.bfloat16),
            pltpu.SemaphoreType.DMA(((N_DEV - 1) * CHUNKS,)),
            pltpu.SemaphoreType.DMA(((N_DEV - 1) * CHUNKS,)),
        ],
        compiler_params=pltpu.CompilerParams(
            collective_id=0, vmem_limit_bytes=100 * 1024 * 1024
        ),
    )(x, dt, slot[:, None], expert_W)


# device time: 51622 ns/iter; 3.9357x vs baseline; 1.0771x over previous
import jax
import jax.numpy as jnp
from jax import lax
from jax.experimental import pallas as pl
from jax.experimental.pallas import tpu as pltpu

N_DEV = 4
N_TOK = 2048
D_IN = 512
D_OUT = 1024
N_EXP = 32
EXP_PER_DEV = N_EXP // N_DEV
CAP = 51
SLOTS = 64
ROWS_PER_DEV = EXP_PER_DEV * SLOTS
CHUNKS = 2
CHUNK_ROWS = ROWS_PER_DEV // CHUNKS
EXP_PER_CHUNK = EXP_PER_DEV // CHUNKS


def kernel(x, router_W, route_idx, expert_W):
    e = route_idx[:, 0].astype(jnp.int32)
    onehot = (e[:, None] == jnp.arange(N_EXP, dtype=jnp.int32)).astype(
        jnp.bfloat16
    )
    tri = jnp.tri(N_TOK, dtype=jnp.bfloat16)
    pos = jnp.dot(tri, onehot, preferred_element_type=jnp.float32)
    my_pos = jnp.sum(pos * onehot.astype(jnp.float32), axis=1).astype(
        jnp.int32
    )
    kept = my_pos <= CAP
    slot = jnp.where(kept, e * SLOTS + (my_pos - 1), -1)

    my = lax.axis_index("i")
    local = kept & (e >= my * EXP_PER_DEV) & (e < (my + 1) * EXP_PER_DEV)
    lslot = jnp.where(local, slot - my * ROWS_PER_DEV, -1)

    def body(x_ref, lslot_ref, slot_ref, w_ref, out_ref, comm_ref,
             send_sems, recv_sems):
        me = lax.axis_index("i")
        row0 = me * ROWS_PER_DEV

        barrier = pltpu.get_barrier_semaphore()
        for k in range(1, N_DEV):
            pl.semaphore_signal(
                barrier, inc=1, device_id=(lax.rem(me + k, N_DEV),),
                device_id_type=pl.DeviceIdType.MESH,
            )
        pl.semaphore_wait(barrier, N_DEV - 1)

        xbf = x_ref[:, :].astype(jnp.bfloat16)
        colc = lax.broadcasted_iota(jnp.int32, (N_TOK, CHUNK_ROWS), 1)

        rdmas = []
        for c in range(CHUNKS):
            dt = (lslot_ref[:, :] == colc + c * CHUNK_ROWS).astype(
                jnp.bfloat16
            )
            cx = lax.dot_general(
                dt, xbf, (((0,), (0,)), ((), ())),
                preferred_element_type=jnp.float32,
            ).astype(jnp.bfloat16)
            parts = [
                jnp.dot(
                    cx[i * SLOTS:(i + 1) * SLOTS, :],
                    w_ref[c * EXP_PER_CHUNK + i].astype(jnp.bfloat16),
                    preferred_element_type=jnp.float32,
                ).astype(jnp.bfloat16)
                for i in range(EXP_PER_CHUNK)
            ]
            chunk = jnp.concatenate(parts, axis=0)
            src = pl.ds(row0 + c * CHUNK_ROWS, CHUNK_ROWS)
            comm_ref[src, :] = chunk
            for k in range(1, N_DEV):
                idx = (k - 1) * CHUNKS + c
                rdma = pltpu.make_async_remote_copy(
                    src_ref=comm_ref.at[src, :],
                    dst_ref=comm_ref.at[src, :],
                    send_sem=send_sems.at[idx],
                    recv_sem=recv_sems.at[idx],
                    device_id=(lax.rem(me + k, N_DEV),),
                    device_id_type=pl.DeviceIdType.MESH,
                )
                rdma.start()
                rdmas.append(rdma)

        def pdot(r0):
            pblk = (slot_ref[:, :] == colc + r0).astype(jnp.bfloat16)
            cblk = comm_ref[pl.ds(r0, CHUNK_ROWS), :]
            return jnp.dot(pblk, cblk, preferred_element_type=jnp.float32)

        acc = pdot(row0)
        for c in range(1, CHUNKS):
            acc = acc + pdot(row0 + c * CHUNK_ROWS)
        for c in range(CHUNKS):
            for k in range(1, N_DEV):
                rdmas[c * (N_DEV - 1) + (k - 1)].wait()
                m = lax.rem(me - k + N_DEV, N_DEV)
                acc = acc + pdot(m * ROWS_PER_DEV + c * CHUNK_ROWS)
        out_ref[:, :] = acc.astype(jnp.bfloat16)

    return pl.pallas_call(
        body,
        out_shape=jax.ShapeDtypeStruct((N_TOK, D_OUT), jnp.bfloat16),
        in_specs=[
            pl.BlockSpec(memory_space=pltpu.VMEM),
            pl.BlockSpec(memory_space=pltpu.VMEM),
            pl.BlockSpec(memory_space=pltpu.VMEM),
            pl.BlockSpec(memory_space=pltpu.VMEM),
        ],
        out_specs=pl.BlockSpec(memory_space=pltpu.VMEM),
        scratch_shapes=[
            pltpu.VMEM((N_DEV * ROWS_PER_DEV, D_OUT), jnp.bfloat16),
            pltpu.SemaphoreType.DMA(((N_DEV - 1) * CHUNKS,)),
            pltpu.SemaphoreType.DMA(((N_DEV - 1) * CHUNKS,)),
        ],
        compiler_params=pltpu.CompilerParams(
            collective_id=0, vmem_limit_bytes=100 * 1024 * 1024
        ),
    )(x, lslot[:, None], slot[:, None], expert_W)


# device time: 49384 ns/iter; 4.1141x vs baseline; 1.0453x over previous
import jax
import jax.numpy as jnp
from jax import lax
from jax.experimental import pallas as pl
from jax.experimental.pallas import tpu as pltpu

N_DEV = 4
N_TOK = 2048
D_IN = 512
D_OUT = 1024
N_EXP = 32
EXP_PER_DEV = N_EXP // N_DEV
CAP = 51
CAP_SEND = 56
SLOTS = 64
ROWS_PER_DEV = EXP_PER_DEV * SLOTS
CHUNKS = 2
CHUNK_ROWS = ROWS_PER_DEV // CHUNKS
EXP_PER_CHUNK = EXP_PER_DEV // CHUNKS


def kernel(x, router_W, route_idx, expert_W):
    e = route_idx[:, 0].astype(jnp.int32)
    onehot = (e[:, None] == jnp.arange(N_EXP, dtype=jnp.int32)).astype(
        jnp.bfloat16
    )
    tri = jnp.tri(N_TOK, dtype=jnp.bfloat16)
    pos = jnp.dot(tri, onehot, preferred_element_type=jnp.float32)
    my_pos = jnp.sum(pos * onehot.astype(jnp.float32), axis=1).astype(
        jnp.int32
    )
    kept = my_pos <= CAP
    slot = jnp.where(kept, e * SLOTS + (my_pos - 1), -1)

    my = lax.axis_index("i")
    local = kept & (e >= my * EXP_PER_DEV) & (e < (my + 1) * EXP_PER_DEV)
    lslot = jnp.where(local, slot - my * ROWS_PER_DEV, -1)

    def body(x_ref, lslot_ref, slot_ref, w_ref, out_ref, comm_ref,
             send_sems, recv_sems):
        me = lax.axis_index("i")
        row0 = me * ROWS_PER_DEV

        comm_ref[:, :] = jnp.zeros((N_DEV * ROWS_PER_DEV, D_OUT), jnp.bfloat16)

        barrier = pltpu.get_barrier_semaphore()
        for k in range(1, N_DEV):
            pl.semaphore_signal(
                barrier, inc=1, device_id=(lax.rem(me + k, N_DEV),),
                device_id_type=pl.DeviceIdType.MESH,
            )
        pl.semaphore_wait(barrier, N_DEV - 1)

        xbf = x_ref[:, :].astype(jnp.bfloat16)
        colc = lax.broadcasted_iota(jnp.int32, (N_TOK, CHUNK_ROWS), 1)

        rdmas = []
        for c in range(CHUNKS):
            dt = (lslot_ref[:, :] == colc + c * CHUNK_ROWS).astype(
                jnp.bfloat16
            )
            cx = lax.dot_general(
                dt, xbf, (((0,), (0,)), ((), ())),
                preferred_element_type=jnp.float32,
            ).astype(jnp.bfloat16)
            parts = [
                jnp.dot(
                    cx[i * SLOTS:(i + 1) * SLOTS, :],
                    w_ref[c * EXP_PER_CHUNK + i].astype(jnp.bfloat16),
                    preferred_element_type=jnp.float32,
                ).astype(jnp.bfloat16)
                for i in range(EXP_PER_CHUNK)
            ]
            chunk = jnp.concatenate(parts, axis=0)
            src = pl.ds(row0 + c * CHUNK_ROWS, CHUNK_ROWS)
            comm_ref[src, :] = chunk
            for k in range(1, N_DEV):
                for i in range(EXP_PER_CHUNK):
                    erow = pl.ds(row0 + c * CHUNK_ROWS + i * SLOTS, CAP_SEND)
                    idx = ((k - 1) * CHUNKS + c) * EXP_PER_CHUNK + i
                    rdma = pltpu.make_async_remote_copy(
                        src_ref=comm_ref.at[erow, :],
                        dst_ref=comm_ref.at[erow, :],
                        send_sem=send_sems.at[idx],
                        recv_sem=recv_sems.at[idx],
                        device_id=(lax.rem(me + k, N_DEV),),
                        device_id_type=pl.DeviceIdType.MESH,
                    )
                    rdma.start()
                    rdmas.append(rdma)

        def pdot(r0):
            pblk = (slot_ref[:, :] == colc + r0).astype(jnp.bfloat16)
            cblk = comm_ref[pl.ds(r0, CHUNK_ROWS), :]
            return jnp.dot(pblk, cblk, preferred_element_type=jnp.float32)

        acc = pdot(row0)
        for c in range(1, CHUNKS):
            acc = acc + pdot(row0 + c * CHUNK_ROWS)
        for c in range(CHUNKS):
            for k in range(1, N_DEV):
                for i in range(EXP_PER_CHUNK):
                    rdmas[
                        (c * (N_DEV - 1) + (k - 1)) * EXP_PER_CHUNK + i
                    ].wait()
                m = lax.rem(me - k + N_DEV, N_DEV)
                acc = acc + pdot(m * ROWS_PER_DEV + c * CHUNK_ROWS)
        out_ref[:, :] = acc.astype(jnp.bfloat16)

    return pl.pallas_call(
        body,
        out_shape=jax.ShapeDtypeStruct((N_TOK, D_OUT), jnp.bfloat16),
        in_specs=[
            pl.BlockSpec(memory_space=pltpu.VMEM),
            pl.BlockSpec(memory_space=pltpu.VMEM),
            pl.BlockSpec(memory_space=pltpu.VMEM),
            pl.BlockSpec(memory_space=pltpu.VMEM),
        ],
        out_specs=pl.BlockSpec(memory_space=pltpu.VMEM),
        scratch_shapes=[
            pltpu.VMEM((N_DEV * ROWS_PER_DEV, D_OUT), jnp.bfloat16),
            pltpu.SemaphoreType.DMA(((N_DEV - 1) * CHUNKS * EXP_PER_CHUNK,)),
            pltpu.SemaphoreType.DMA(((N_DEV - 1) * CHUNKS * EXP_PER_CHUNK,)),
        ],
        compiler_params=pltpu.CompilerParams(
            collective_id=0, vmem_limit_bytes=100 * 1024 * 1024
        ),
    )(x, lslot[:, None], slot[:, None], expert_W)


# device time: 46161 ns/iter; 4.4013x vs baseline; 1.0698x over previous
import jax
import jax.numpy as jnp
from jax import lax
from jax.experimental import pallas as pl
from jax.experimental.pallas import tpu as pltpu

N_DEV = 4
N_TOK = 2048
D_IN = 512
D_OUT = 1024
N_EXP = 32
EXP_PER_DEV = N_EXP // N_DEV
CAP = 51
CAP_SEND = 56
SLOTS = 64
ROWS_PER_DEV = EXP_PER_DEV * SLOTS
CHUNKS = 2
CHUNK_ROWS = ROWS_PER_DEV // CHUNKS
EXP_PER_CHUNK = EXP_PER_DEV // CHUNKS


def kernel(x, router_W, route_idx, expert_W):
    e = route_idx[:, 0].astype(jnp.int32)
    onehot = (e[:, None] == jnp.arange(N_EXP, dtype=jnp.int32)).astype(
        jnp.bfloat16
    )
    tri = jnp.tri(N_TOK, dtype=jnp.bfloat16)
    pos = jnp.dot(tri, onehot, preferred_element_type=jnp.float32)
    my_pos = jnp.sum(pos * onehot.astype(jnp.float32), axis=1).astype(
        jnp.int32
    )
    kept = my_pos <= CAP
    slot = jnp.where(kept, e * SLOTS + (my_pos - 1), -1)

    my = lax.axis_index("i")
    local = kept & (e >= my * EXP_PER_DEV) & (e < (my + 1) * EXP_PER_DEV)
    lslot = jnp.where(local, slot - my * ROWS_PER_DEV, -1)

    def body(x_ref, lslot_ref, slot_ref, w_hbm, out_ref, comm_ref,
             w_ref, w_sems, send_sems, recv_sems):
        me = lax.axis_index("i")
        row0 = me * ROWS_PER_DEV

        w_copies = []
        for c in range(CHUNKS):
            cp = pltpu.make_async_copy(
                w_hbm.at[pl.ds(c * EXP_PER_CHUNK, EXP_PER_CHUNK)],
                w_ref.at[pl.ds(c * EXP_PER_CHUNK, EXP_PER_CHUNK)],
                w_sems.at[c],
            )
            cp.start()
            w_copies.append(cp)

        comm_ref[:, :] = jnp.zeros((N_DEV * ROWS_PER_DEV, D_OUT), jnp.bfloat16)

        barrier = pltpu.get_barrier_semaphore()
        for k in range(1, N_DEV):
            pl.semaphore_signal(
                barrier, inc=1, device_id=(lax.rem(me + k, N_DEV),),
                device_id_type=pl.DeviceIdType.MESH,
            )
        pl.semaphore_wait(barrier, N_DEV - 1)

        xbf = x_ref[:, :].astype(jnp.bfloat16)
        colc = lax.broadcasted_iota(jnp.int32, (N_TOK, CHUNK_ROWS), 1)

        rdmas = []
        for c in range(CHUNKS):
            dt = (lslot_ref[:, :] == colc + c * CHUNK_ROWS).astype(
                jnp.bfloat16
            )
            cx = lax.dot_general(
                dt, xbf, (((0,), (0,)), ((), ())),
                preferred_element_type=jnp.float32,
            ).astype(jnp.bfloat16)
            w_copies[c].wait()
            parts = [
                jnp.dot(
                    cx[i * SLOTS:(i + 1) * SLOTS, :],
                    w_ref[c * EXP_PER_CHUNK + i].astype(jnp.bfloat16),
                    preferred_element_type=jnp.float32,
                ).astype(jnp.bfloat16)
                for i in range(EXP_PER_CHUNK)
            ]
            chunk = jnp.concatenate(parts, axis=0)
            src = pl.ds(row0 + c * CHUNK_ROWS, CHUNK_ROWS)
            comm_ref[src, :] = chunk
            for k in range(1, N_DEV):
                for i in range(EXP_PER_CHUNK):
                    erow = pl.ds(row0 + c * CHUNK_ROWS + i * SLOTS, CAP_SEND)
                    idx = ((k - 1) * CHUNKS + c) * EXP_PER_CHUNK + i
                    rdma = pltpu.make_async_remote_copy(
                        src_ref=comm_ref.at[erow, :],
                        dst_ref=comm_ref.at[erow, :],
                        send_sem=send_sems.at[idx],
                        recv_sem=recv_sems.at[idx],
                        device_id=(lax.rem(me + k, N_DEV),),
                        device_id_type=pl.DeviceIdType.MESH,
                    )
                    rdma.start()
                    rdmas.append(rdma)

        def pdot(r0):
            pblk = (slot_ref[:, :] == colc + r0).astype(jnp.bfloat16)
            cblk = comm_ref[pl.ds(r0, CHUNK_ROWS), :]
            return jnp.dot(pblk, cblk, preferred_element_type=jnp.float32)

        acc = pdot(row0)
        for c in range(1, CHUNKS):
            acc = acc + pdot(row0 + c * CHUNK_ROWS)
        for c in range(CHUNKS):
            for k in range(1, N_DEV):
                for i in range(EXP_PER_CHUNK):
                    rdmas[
                        (c * (N_DEV - 1) + (k - 1)) * EXP_PER_CHUNK + i
                    ].wait()
                m = lax.rem(me - k + N_DEV, N_DEV)
                acc = acc + pdot(m * ROWS_PER_DEV + c * CHUNK_ROWS)
        out_ref[:, :] = acc.astype(jnp.bfloat16)

    return pl.pallas_call(
        body,
        out_shape=jax.ShapeDtypeStruct((N_TOK, D_OUT), jnp.bfloat16),
        in_specs=[
            pl.BlockSpec(memory_space=pltpu.VMEM),
            pl.BlockSpec(memory_space=pltpu.VMEM),
            pl.BlockSpec(memory_space=pltpu.VMEM),
            pl.BlockSpec(memory_space=pl.ANY),
        ],
        out_specs=pl.BlockSpec(memory_space=pltpu.VMEM),
        scratch_shapes=[
            pltpu.VMEM((N_DEV * ROWS_PER_DEV, D_OUT), jnp.bfloat16),
            pltpu.VMEM((EXP_PER_DEV, D_IN, D_OUT), jnp.float32),
            pltpu.SemaphoreType.DMA((CHUNKS,)),
            pltpu.SemaphoreType.DMA(((N_DEV - 1) * CHUNKS * EXP_PER_CHUNK,)),
            pltpu.SemaphoreType.DMA(((N_DEV - 1) * CHUNKS * EXP_PER_CHUNK,)),
        ],
        compiler_params=pltpu.CompilerParams(
            collective_id=0, vmem_limit_bytes=100 * 1024 * 1024
        ),
    )(x, lslot[:, None], slot[:, None], expert_W)
